# baseline (speedup 1.0000x reference)
import functools

import jax
import jax.numpy as jnp
from jax import lax
from jax.experimental import pallas as pl
from jax.experimental.pallas import tpu as pltpu
from jax.experimental.pallas import tpu_sc as plsc

_BATCH = 16384
_D = 64
_NC = 2
_NS = 16
_NW = _NC * _NS
_BPW = _BATCH // _NW
_CH = 128
_NCH = _BPW // _CH
_L = 16


def _tile_body(feat_hbm, lab_hbm, cent_hbm, out_hbm,
               lab_v, idx_v, rows_v, feat_v, part_v, gsem, fsem):
    wid = lax.axis_index("s") * _NC + lax.axis_index("c")
    base = wid * _BPW

    pltpu.sync_copy(lab_hbm.at[pl.ds(base, _BPW)], lab_v)

    fcopy = pltpu.make_async_copy(
        feat_hbm.at[pl.ds(wid * (_BPW // 2), _BPW // 2)], feat_v, fsem)
    fcopy.start()

    def mkidx(j, _):
        v = lab_v[pl.ds(j * _L, _L)]
        idx_v[j // (_CH // _L), pl.ds((j % (_CH // _L)) * _L, _L)] = (
            lax.shift_right_logical(v, 1))
        return 0

    def mkidx_c(c):
        for jj in range(_CH // _L):
            v = lab_v[pl.ds(c * _CH + jj * _L, _L)]
            idx_v[c, pl.ds(jj * _L, _L)] = lax.shift_right_logical(v, 1)

    gathers = []
    for c in range(_NCH):
        mkidx_c(c)
        g = pltpu.make_async_copy(
            cent_hbm.at[idx_v.at[c]],
            rows_v.at[pl.ds(c * _CH, _CH)],
            gsem,
        )
        g.start()
        gathers.append(g)
    for g in gathers:
        g.wait()
    fcopy.wait()

    zero = jnp.zeros((_L,), jnp.float32)

    def body(g, accs):
        a0, a1, a2, a3 = accs
        v = lab_v[pl.ds(g * _L, _L)]
        for j in range(_L):
            i = g * _L + j
            p = (v[j] & 1) * _D
            fr = g * (_L // 2) + (j // 2)
            fo = (j % 2) * _D
            d0 = feat_v[fr, pl.ds(fo, _L)] - rows_v[i, pl.ds(p, _L)]
            d1 = feat_v[fr, pl.ds(fo + _L, _L)] - rows_v[i, pl.ds(p + _L, _L)]
            d2 = (feat_v[fr, pl.ds(fo + 2 * _L, _L)]
                  - rows_v[i, pl.ds(p + 2 * _L, _L)])
            d3 = (feat_v[fr, pl.ds(fo + 3 * _L, _L)]
                  - rows_v[i, pl.ds(p + 3 * _L, _L)])
            a0 = a0 + d0 * d0
            a1 = a1 + d1 * d1
            a2 = a2 + d2 * d2
            a3 = a3 + d3 * d3
        return (a0, a1, a2, a3)

    a0, a1, a2, a3 = lax.fori_loop(0, _BPW // _L, body, (zero,) * 4)
    part_v[...] = (a0 + a1) + (a2 + a3)
    pltpu.sync_copy(part_v, out_hbm.at[pl.ds(wid * _L, _L)])


_mesh = plsc.VectorSubcoreMesh(core_axis_name="c", subcore_axis_name="s")

_sc_call = functools.partial(
    pl.kernel,
    mesh=_mesh,
    out_type=jax.ShapeDtypeStruct((_NW * _L,), jnp.float32),
    scratch_types=[
        pltpu.VMEM((_BPW,), jnp.int32),
        pltpu.VMEM((_NCH, _CH), jnp.int32),
        pltpu.VMEM((_BPW, 2 * _D), jnp.float32),
        pltpu.VMEM((_BPW // 2, 2 * _D), jnp.float32),
        pltpu.VMEM((_L,), jnp.float32),
        pltpu.SemaphoreType.DMA,
        pltpu.SemaphoreType.DMA,
    ],
    compiler_params=pltpu.CompilerParams(use_tc_tiling_on_sc=True),
)(_tile_body)


def kernel(features, labels, centers):
    lab = labels.astype(jnp.int32)
    feat2 = features.reshape(_BATCH // 2, 2 * _D)
    cent2 = centers.reshape(-1, 2 * _D)
    partials = _sc_call(feat2, lab, cent2)
    return jnp.sum(partials) * 0.5


# speedup vs baseline: 1.0092x; 1.0092x over previous
import functools

import jax
import jax.numpy as jnp
from jax import lax
from jax.experimental import pallas as pl
from jax.experimental.pallas import tpu as pltpu
from jax.experimental.pallas import tpu_sc as plsc

_BATCH = 16384
_D = 64
_NC = 2
_NS = 16
_NW = _NC * _NS
_BPW = _BATCH // _NW
_CH = 128
_NCH = _BPW // _CH
_L = 16


def _tile_body(feat_hbm, lab_hbm, cent_hbm, out_hbm,
               lab_v, idx_v, rows_v, feat_v, part_v, gsem, fsem):
    wid = lax.axis_index("s") * _NC + lax.axis_index("c")
    base = wid * _BPW

    pltpu.sync_copy(lab_hbm.at[pl.ds(base, _BPW)], lab_v)

    fcopy = pltpu.make_async_copy(
        feat_hbm.at[pl.ds(base * _D, _BPW * _D)], feat_v, fsem)
    fcopy.start()

    gathers = []
    for c in range(_NCH):
        for jj in range(_CH // _L):
            v = lab_v[pl.ds(c * _CH + jj * _L, _L)]
            idx_v[c, pl.ds(jj * _L, _L)] = lax.shift_right_logical(v, 1)
        g = pltpu.make_async_copy(
            cent_hbm.at[idx_v.at[c]],
            rows_v.at[pl.ds(c * _CH, _CH)],
            gsem,
        )
        g.start()
        gathers.append(g)
    for g in gathers:
        g.wait()
    fcopy.wait()

    zero = jnp.zeros((_L,), jnp.float32)

    def body(g, accs):
        a0, a1, a2, a3 = accs
        v = lab_v[pl.ds(g * _L, _L)]
        for j in range(_L):
            i = g * _L + j
            p = (v[j] & 1) * _D
            fb = i * _D
            d0 = feat_v[pl.ds(fb, _L)] - rows_v[i, pl.ds(p, _L)]
            d1 = feat_v[pl.ds(fb + _L, _L)] - rows_v[i, pl.ds(p + _L, _L)]
            d2 = (feat_v[pl.ds(fb + 2 * _L, _L)]
                  - rows_v[i, pl.ds(p + 2 * _L, _L)])
            d3 = (feat_v[pl.ds(fb + 3 * _L, _L)]
                  - rows_v[i, pl.ds(p + 3 * _L, _L)])
            a0 = a0 + d0 * d0
            a1 = a1 + d1 * d1
            a2 = a2 + d2 * d2
            a3 = a3 + d3 * d3
        return (a0, a1, a2, a3)

    a0, a1, a2, a3 = lax.fori_loop(0, _BPW // _L, body, (zero,) * 4)
    part_v[...] = (a0 + a1) + (a2 + a3)
    pltpu.sync_copy(part_v, out_hbm.at[pl.ds(wid * _L, _L)])


_mesh = plsc.VectorSubcoreMesh(core_axis_name="c", subcore_axis_name="s")

_sc_call = functools.partial(
    pl.kernel,
    mesh=_mesh,
    out_type=jax.ShapeDtypeStruct((_NW * _L,), jnp.float32),
    scratch_types=[
        pltpu.VMEM((_BPW,), jnp.int32),
        pltpu.VMEM((_NCH, _CH), jnp.int32),
        pltpu.VMEM((_BPW, 2 * _D), jnp.float32),
        pltpu.VMEM((_BPW * _D,), jnp.float32),
        pltpu.VMEM((_L,), jnp.float32),
        pltpu.SemaphoreType.DMA,
        pltpu.SemaphoreType.DMA,
    ],
)(_tile_body)


def kernel(features, labels, centers):
    lab = labels.astype(jnp.int32)
    feat1 = features.reshape(-1)
    cent2 = centers.reshape(-1, 2 * _D)
    partials = _sc_call(feat1, lab, cent2)
    return jnp.sum(partials) * 0.5


# speedup vs baseline: 2.2292x; 2.2090x over previous
import functools

import jax
import jax.numpy as jnp
from jax import lax
from jax.experimental import pallas as pl
from jax.experimental.pallas import tpu as pltpu
from jax.experimental.pallas import tpu_sc as plsc

_BATCH = 16384
_D = 64
_V = 100000
_NC = 2
_NS = 16
_NW = _NC * _NS
_DPW = _D // _NW
_L = 16
_FCH = 2048
_NFCH = _BATCH // _FCH


def _tile_body(feat_hbm, lab_hbm, cent_hbm, out_hbm,
               lab_v, row_v, fbuf, part_v, rsem, fsem):
    wid = lax.axis_index("s") * _NC + lax.axis_index("c")

    pltpu.sync_copy(lab_hbm, lab_v)

    zero = jnp.zeros((_L,), jnp.float32)
    accs = (zero, zero, zero, zero)

    for di in range(_DPW):
        d = wid + di * _NW

        rcopy = pltpu.make_async_copy(cent_hbm.at[d], row_v, rsem)
        rcopy.start()

        f0 = pltpu.make_async_copy(
            feat_hbm.at[d, pl.ds(0, _FCH)], fbuf.at[0], fsem)
        f0.start()
        rcopy.wait()

        def chunk2(k2, accs, d=d):
            for b in range(2):
                k = k2 * 2 + b

                @pl.when(k + 1 < _NFCH)
                def _(k=k, b=b, d=d):
                    pltpu.make_async_copy(
                        feat_hbm.at[d, pl.ds((k + 1) * _FCH, _FCH)],
                        fbuf.at[(b + 1) % 2], fsem).start()

                pltpu.make_async_copy(
                    feat_hbm.at[d, pl.ds(0, _FCH)], fbuf.at[b], fsem).wait()

                def grp(j, accs, k=k, b=b):
                    a0, a1, a2, a3 = accs
                    base = k * _FCH + j * 4 * _L
                    idx0 = lab_v[pl.ds(base, _L)]
                    idx1 = lab_v[pl.ds(base + _L, _L)]
                    idx2 = lab_v[pl.ds(base + 2 * _L, _L)]
                    idx3 = lab_v[pl.ds(base + 3 * _L, _L)]
                    g0 = plsc.load_gather(row_v, [idx0])
                    g1 = plsc.load_gather(row_v, [idx1])
                    g2 = plsc.load_gather(row_v, [idx2])
                    g3 = plsc.load_gather(row_v, [idx3])
                    fb = j * 4 * _L
                    d0 = fbuf[b, pl.ds(fb, _L)] - g0
                    d1 = fbuf[b, pl.ds(fb + _L, _L)] - g1
                    d2 = fbuf[b, pl.ds(fb + 2 * _L, _L)] - g2
                    d3 = fbuf[b, pl.ds(fb + 3 * _L, _L)] - g3
                    return (a0 + d0 * d0, a1 + d1 * d1,
                            a2 + d2 * d2, a3 + d3 * d3)

                accs = lax.fori_loop(0, _FCH // (4 * _L), grp, accs)
            return accs

        accs = lax.fori_loop(0, _NFCH // 2, chunk2, accs)

    a0, a1, a2, a3 = accs
    part_v[...] = (a0 + a1) + (a2 + a3)
    pltpu.sync_copy(part_v, out_hbm.at[pl.ds(wid * _L, _L)])


_mesh = plsc.VectorSubcoreMesh(core_axis_name="c", subcore_axis_name="s")

_sc_call = functools.partial(
    pl.kernel,
    mesh=_mesh,
    compiler_params=pltpu.CompilerParams(needs_layout_passes=False),
    out_type=jax.ShapeDtypeStruct((_NW * _L,), jnp.float32),
    scratch_types=[
        pltpu.VMEM((_BATCH,), jnp.int32),
        pltpu.VMEM((_V,), jnp.float32),
        pltpu.VMEM((2, _FCH), jnp.float32),
        pltpu.VMEM((_L,), jnp.float32),
        pltpu.SemaphoreType.DMA,
        pltpu.SemaphoreType.DMA,
    ],
)(_tile_body)


def kernel(features, labels, centers):
    lab = labels.astype(jnp.int32)
    partials = _sc_call(features.T, lab, centers.T)
    return jnp.sum(partials) * 0.5


# speedup vs baseline: 2.3119x; 1.0371x over previous
import functools

import jax
import jax.numpy as jnp
from jax import lax
from jax.experimental import pallas as pl
from jax.experimental.pallas import tpu as pltpu
from jax.experimental.pallas import tpu_sc as plsc

_BATCH = 16384
_D = 64
_V = 100000
_NC = 2
_NS = 16
_NW = _NC * _NS
_DPW = _D // _NW
_L = 16
_FCH = 4096


def _tile_body(feat_hbm, lab_hbm, cent_hbm, out_hbm,
               lab_v, row_v, fbuf, part_v, rsem, fsem):
    wid = lax.axis_index("s") * _NC + lax.axis_index("c")

    pltpu.sync_copy(lab_hbm, lab_v)

    zero = jnp.zeros((_L,), jnp.float32)
    accs = (zero, zero, zero, zero)

    for di in range(_DPW):
        d = wid + di * _NW

        rcopy = pltpu.make_async_copy(cent_hbm.at[d], row_v, rsem)
        rcopy.start()

        for q in range(2):
            pltpu.make_async_copy(
                feat_hbm.at[d, pl.ds(q * _FCH, _FCH)], fbuf.at[q],
                fsem).start()
        rcopy.wait()

        for q in range(4):
            pltpu.make_async_copy(
                feat_hbm.at[d, pl.ds(q * _FCH, _FCH)], fbuf.at[q % 2],
                fsem).wait()

            def grp(j, accs, q=q):
                a0, a1, a2, a3 = accs
                lb = q * _FCH + j * 4 * _L
                fb = j * 4 * _L
                idx0 = lab_v[pl.ds(lb, _L)]
                idx1 = lab_v[pl.ds(lb + _L, _L)]
                idx2 = lab_v[pl.ds(lb + 2 * _L, _L)]
                idx3 = lab_v[pl.ds(lb + 3 * _L, _L)]
                g0 = plsc.load_gather(row_v, [idx0])
                g1 = plsc.load_gather(row_v, [idx1])
                g2 = plsc.load_gather(row_v, [idx2])
                g3 = plsc.load_gather(row_v, [idx3])
                d0 = fbuf[q % 2, pl.ds(fb, _L)] - g0
                d1 = fbuf[q % 2, pl.ds(fb + _L, _L)] - g1
                d2 = fbuf[q % 2, pl.ds(fb + 2 * _L, _L)] - g2
                d3 = fbuf[q % 2, pl.ds(fb + 3 * _L, _L)] - g3
                return (a0 + d0 * d0, a1 + d1 * d1,
                        a2 + d2 * d2, a3 + d3 * d3)

            accs = lax.fori_loop(0, _FCH // (4 * _L), grp, accs)
            if q + 2 < 4:
                pltpu.make_async_copy(
                    feat_hbm.at[d, pl.ds((q + 2) * _FCH, _FCH)],
                    fbuf.at[q % 2], fsem).start()

    a0, a1, a2, a3 = accs
    part_v[...] = (a0 + a1) + (a2 + a3)
    pltpu.sync_copy(part_v, out_hbm.at[pl.ds(wid * _L, _L)])


_mesh = plsc.VectorSubcoreMesh(core_axis_name="c", subcore_axis_name="s")

_sc_call = functools.partial(
    pl.kernel,
    mesh=_mesh,
    compiler_params=pltpu.CompilerParams(needs_layout_passes=False),
    out_type=jax.ShapeDtypeStruct((_NW * _L,), jnp.float32),
    scratch_types=[
        pltpu.VMEM((_BATCH,), jnp.int32),
        pltpu.VMEM((_V,), jnp.float32),
        pltpu.VMEM((2, _FCH), jnp.float32),
        pltpu.VMEM((_L,), jnp.float32),
        pltpu.SemaphoreType.DMA,
        pltpu.SemaphoreType.DMA,
    ],
)(_tile_body)


def kernel(features, labels, centers):
    lab = labels.astype(jnp.int32)
    partials = _sc_call(features.T, lab, centers.T)
    return jnp.sum(partials) * 0.5


# speedup vs baseline: 2.3723x; 1.0261x over previous
import functools

import jax
import jax.numpy as jnp
from jax import lax
from jax.experimental import pallas as pl
from jax.experimental.pallas import tpu as pltpu
from jax.experimental.pallas import tpu_sc as plsc

_BATCH = 16384
_D = 64
_V = 100000
_NC = 2
_NS = 16
_NW = _NC * _NS
_DPW = _D // _NW
_L = 16
_FCH = 4096


def _tile_body(feat_hbm, lab_hbm, cent_hbm, out_hbm,
               lab_v, row_v, fbuf, part_v, rsem, fsem):
    wid = lax.axis_index("s") * _NC + lax.axis_index("c")

    pltpu.sync_copy(lab_hbm, lab_v)

    zero = jnp.zeros((_L,), jnp.float32)
    accs = (zero, zero, zero, zero)

    def dim_body(di, accs):
        d = wid + di * _NW

        rcopy = pltpu.make_async_copy(cent_hbm.at[d], row_v, rsem)
        rcopy.start()

        for q in range(2):
            pltpu.make_async_copy(
                feat_hbm.at[d, pl.ds(q * _FCH, _FCH)],
                fbuf.at[pl.ds(q * _FCH, _FCH)], fsem).start()
        rcopy.wait()

        def quarter(q, accs):
            slot = (q % 2) * _FCH
            pltpu.make_async_copy(
                feat_hbm.at[d, pl.ds(q * _FCH, _FCH)],
                fbuf.at[pl.ds(slot, _FCH)], fsem).wait()

            def grp(j, accs):
                a0, a1, a2, a3 = accs
                lb = q * _FCH + j * 4 * _L
                fb = slot + j * 4 * _L
                idx0 = lab_v[pl.ds(lb, _L)]
                idx1 = lab_v[pl.ds(lb + _L, _L)]
                idx2 = lab_v[pl.ds(lb + 2 * _L, _L)]
                idx3 = lab_v[pl.ds(lb + 3 * _L, _L)]
                g0 = plsc.load_gather(row_v, [idx0])
                g1 = plsc.load_gather(row_v, [idx1])
                g2 = plsc.load_gather(row_v, [idx2])
                g3 = plsc.load_gather(row_v, [idx3])
                d0 = fbuf[pl.ds(fb, _L)] - g0
                d1 = fbuf[pl.ds(fb + _L, _L)] - g1
                d2 = fbuf[pl.ds(fb + 2 * _L, _L)] - g2
                d3 = fbuf[pl.ds(fb + 3 * _L, _L)] - g3
                return (a0 + d0 * d0, a1 + d1 * d1,
                        a2 + d2 * d2, a3 + d3 * d3)

            accs = lax.fori_loop(0, _FCH // (4 * _L), grp, accs)

            @pl.when(q + 2 < 4)
            def _():
                pltpu.make_async_copy(
                    feat_hbm.at[d, pl.ds((q + 2) * _FCH, _FCH)],
                    fbuf.at[pl.ds(slot, _FCH)], fsem).start()

            return accs

        return lax.fori_loop(0, 4, quarter, accs)

    accs = lax.fori_loop(0, _DPW, dim_body, accs)

    a0, a1, a2, a3 = accs
    part_v[...] = (a0 + a1) + (a2 + a3)
    pltpu.sync_copy(part_v, out_hbm.at[pl.ds(wid * _L, _L)])


_mesh = plsc.VectorSubcoreMesh(core_axis_name="c", subcore_axis_name="s")

_sc_call = functools.partial(
    pl.kernel,
    mesh=_mesh,
    compiler_params=pltpu.CompilerParams(needs_layout_passes=False),
    out_type=jax.ShapeDtypeStruct((_NW * _L,), jnp.float32),
    scratch_types=[
        pltpu.VMEM((_BATCH,), jnp.int32),
        pltpu.VMEM((_V,), jnp.float32),
        pltpu.VMEM((2 * _FCH,), jnp.float32),
        pltpu.VMEM((_L,), jnp.float32),
        pltpu.SemaphoreType.DMA,
        pltpu.SemaphoreType.DMA,
    ],
)(_tile_body)


def kernel(features, labels, centers):
    lab = labels.astype(jnp.int32)
    partials = _sc_call(features.T, lab, centers.T)
    return jnp.sum(partials) * 0.5


# speedup vs baseline: 2.4899x; 1.0495x over previous
import functools

import jax
import jax.numpy as jnp
from jax import lax
from jax.experimental import pallas as pl
from jax.experimental.pallas import tpu as pltpu
from jax.experimental.pallas import tpu_sc as plsc

_BATCH = 16384
_D = 64
_V = 100000
_NC = 2
_NS = 16
_NW = _NC * _NS
_DPW = _D // _NW
_L = 16
_FCH = 4096


def _tile_body(feat_hbm, lab_hbm, cent_hbm, out_hbm,
               lab_v, row_v, fbuf, part_v, rsem, fsem):
    wid = lax.axis_index("s") * _NC + lax.axis_index("c")

    pltpu.make_async_copy(cent_hbm.at[wid], row_v, rsem).start()
    pltpu.sync_copy(lab_hbm, lab_v)

    zero = jnp.zeros((_L,), jnp.float32)
    accs = (zero, zero, zero, zero)

    def dim_body(di, accs):
        d = wid + di * _NW

        @pl.when(di > 0)
        def _():
            pltpu.make_async_copy(cent_hbm.at[d], row_v, rsem).start()

        for q in range(2):
            pltpu.make_async_copy(
                feat_hbm.at[d, pl.ds(q * _FCH, _FCH)],
                fbuf.at[pl.ds(q * _FCH, _FCH)], fsem).start()
        pltpu.make_async_copy(cent_hbm.at[d], row_v, rsem).wait()

        def quarter(q, accs):
            slot = (q % 2) * _FCH
            pltpu.make_async_copy(
                feat_hbm.at[d, pl.ds(q * _FCH, _FCH)],
                fbuf.at[pl.ds(slot, _FCH)], fsem).wait()

            def grp(j, accs):
                a0, a1, a2, a3 = accs
                lb = q * _FCH + j * 4 * _L
                fb = slot + j * 4 * _L
                idx0 = lab_v[pl.ds(lb, _L)]
                idx1 = lab_v[pl.ds(lb + _L, _L)]
                idx2 = lab_v[pl.ds(lb + 2 * _L, _L)]
                idx3 = lab_v[pl.ds(lb + 3 * _L, _L)]
                g0 = plsc.load_gather(row_v, [idx0])
                g1 = plsc.load_gather(row_v, [idx1])
                g2 = plsc.load_gather(row_v, [idx2])
                g3 = plsc.load_gather(row_v, [idx3])
                d0 = fbuf[pl.ds(fb, _L)] - g0
                d1 = fbuf[pl.ds(fb + _L, _L)] - g1
                d2 = fbuf[pl.ds(fb + 2 * _L, _L)] - g2
                d3 = fbuf[pl.ds(fb + 3 * _L, _L)] - g3
                return (a0 + d0 * d0, a1 + d1 * d1,
                        a2 + d2 * d2, a3 + d3 * d3)

            accs = lax.fori_loop(0, _FCH // (4 * _L), grp, accs)

            @pl.when(q + 2 < 4)
            def _():
                pltpu.make_async_copy(
                    feat_hbm.at[d, pl.ds((q + 2) * _FCH, _FCH)],
                    fbuf.at[pl.ds(slot, _FCH)], fsem).start()

            return accs

        return lax.fori_loop(0, 4, quarter, accs)

    accs = lax.fori_loop(0, _DPW, dim_body, accs)

    a0, a1, a2, a3 = accs
    part_v[...] = (a0 + a1) + (a2 + a3)
    pltpu.sync_copy(part_v, out_hbm.at[pl.ds(wid * _L, _L)])


_mesh = plsc.VectorSubcoreMesh(core_axis_name="c", subcore_axis_name="s")

_sc_call = functools.partial(
    pl.kernel,
    mesh=_mesh,
    compiler_params=pltpu.CompilerParams(needs_layout_passes=False),
    out_type=jax.ShapeDtypeStruct((_NW * _L,), jnp.float32),
    scratch_types=[
        pltpu.VMEM((_BATCH,), jnp.int32),
        pltpu.VMEM((_V,), jnp.float32),
        pltpu.VMEM((2 * _FCH,), jnp.float32),
        pltpu.VMEM((_L,), jnp.float32),
        pltpu.SemaphoreType.DMA,
        pltpu.SemaphoreType.DMA,
    ],
)(_tile_body)


def kernel(features, labels, centers):
    lab = labels.astype(jnp.int32)
    partials = _sc_call(features.T, lab, centers.T)
    return jnp.sum(partials) * 0.5


# speedup vs baseline: 2.4943x; 1.0018x over previous
import functools

import jax
import jax.numpy as jnp
from jax import lax
from jax.experimental import pallas as pl
from jax.experimental.pallas import tpu as pltpu
from jax.experimental.pallas import tpu_sc as plsc

_BATCH = 16384
_D = 64
_V = 100000
_NC = 2
_NS = 16
_NW = _NC * _NS
_DPW = _D // _NW
_L = 16
_FCH = 4096


def _tile_body(feat_hbm, lab_hbm, cent_hbm, out_hbm,
               lab_v, row_v, fbuf, part_v, rsem, fsem):
    wid = lax.axis_index("s") * _NC + lax.axis_index("c")

    pltpu.make_async_copy(cent_hbm.at[wid], row_v, rsem).start()
    pltpu.sync_copy(lab_hbm, lab_v)

    zero = jnp.zeros((_L,), jnp.float32)
    accs = (zero, zero, zero, zero)

    def dim_body(di, accs):
        d = wid + di * _NW

        @pl.when(di > 0)
        def _():
            pltpu.make_async_copy(cent_hbm.at[d], row_v, rsem).start()

        for q in range(2):
            pltpu.make_async_copy(
                feat_hbm.at[d, pl.ds(q * _FCH, _FCH)],
                fbuf.at[pl.ds(q * _FCH, _FCH)], fsem).start()
        pltpu.make_async_copy(cent_hbm.at[d], row_v, rsem).wait()

        def quarter(q, accs):
            slot = (q % 2) * _FCH
            pltpu.make_async_copy(
                feat_hbm.at[d, pl.ds(q * _FCH, _FCH)],
                fbuf.at[pl.ds(slot, _FCH)], fsem).wait()

            def grp(j, accs):
                a0, a1, a2, a3 = accs
                lb = q * _FCH + j * 4 * _L
                fb = slot + j * 4 * _L
                idx0 = lab_v[pl.ds(lb, _L)]
                idx1 = lab_v[pl.ds(lb + _L, _L)]
                idx2 = lab_v[pl.ds(lb + 2 * _L, _L)]
                idx3 = lab_v[pl.ds(lb + 3 * _L, _L)]
                g0 = plsc.load_gather(row_v, [idx0])
                g1 = plsc.load_gather(row_v, [idx1])
                g2 = plsc.load_gather(row_v, [idx2])
                g3 = plsc.load_gather(row_v, [idx3])
                d0 = fbuf[pl.ds(fb, _L)] - g0
                d1 = fbuf[pl.ds(fb + _L, _L)] - g1
                d2 = fbuf[pl.ds(fb + 2 * _L, _L)] - g2
                d3 = fbuf[pl.ds(fb + 3 * _L, _L)] - g3
                return (a0 + d0 * d0, a1 + d1 * d1,
                        a2 + d2 * d2, a3 + d3 * d3)

            accs = lax.fori_loop(0, _FCH // (4 * _L), grp, accs)

            @pl.when(q + 2 < 4)
            def _():
                pltpu.make_async_copy(
                    feat_hbm.at[d, pl.ds((q + 2) * _FCH, _FCH)],
                    fbuf.at[pl.ds(slot, _FCH)], fsem).start()

            return accs

        return lax.fori_loop(0, 4, quarter, accs)

    accs = lax.fori_loop(0, _DPW, dim_body, accs)

    a0, a1, a2, a3 = accs
    part_v[...] = (a0 + a1) + (a2 + a3)
    pltpu.sync_copy(part_v, out_hbm.at[pl.ds(wid * _L, _L)])


_mesh = plsc.VectorSubcoreMesh(core_axis_name="c", subcore_axis_name="s")

_sc_call = functools.partial(
    pl.kernel,
    mesh=_mesh,
    compiler_params=pltpu.CompilerParams(needs_layout_passes=False, skip_device_barrier=True),
    out_type=jax.ShapeDtypeStruct((_NW * _L,), jnp.float32),
    scratch_types=[
        pltpu.VMEM((_BATCH,), jnp.int32),
        pltpu.VMEM((_V,), jnp.float32),
        pltpu.VMEM((2 * _FCH,), jnp.float32),
        pltpu.VMEM((_L,), jnp.float32),
        pltpu.SemaphoreType.DMA,
        pltpu.SemaphoreType.DMA,
    ],
)(_tile_body)


def kernel(features, labels, centers):
    lab = labels.astype(jnp.int32)
    partials = _sc_call(features.T, lab, centers.T)
    return jnp.sum(partials) * 0.5


# speedup vs baseline: 2.5019x; 1.0030x over previous
import functools

import jax
import jax.numpy as jnp
from jax import lax
from jax.experimental import pallas as pl
from jax.experimental.pallas import tpu as pltpu
from jax.experimental.pallas import tpu_sc as plsc

_BATCH = 16384
_D = 64
_V = 100000
_NC = 2
_NS = 16
_NW = _NC * _NS
_DPW = _D // _NW
_L = 16
_FCH = 4096


def _tile_body(feat_hbm, lab_hbm, cent_hbm, out_hbm,
               lab_v, row_v, fbuf, part_v, rsem, fsem):
    wid = lax.axis_index("s") * _NC + lax.axis_index("c")

    pltpu.make_async_copy(cent_hbm.at[wid], row_v, rsem).start()
    pltpu.sync_copy(lab_hbm, lab_v)

    zero = jnp.zeros((_L,), jnp.float32)
    accs = (zero,) * 8

    def dim_body(di, accs):
        d = wid + di * _NW

        @pl.when(di > 0)
        def _():
            pltpu.make_async_copy(cent_hbm.at[d], row_v, rsem).start()

        for q in range(2):
            pltpu.make_async_copy(
                feat_hbm.at[d, pl.ds(q * _FCH, _FCH)],
                fbuf.at[pl.ds(q * _FCH, _FCH)], fsem).start()
        pltpu.make_async_copy(cent_hbm.at[d], row_v, rsem).wait()

        def quarter(q, accs):
            slot = (q % 2) * _FCH
            pltpu.make_async_copy(
                feat_hbm.at[d, pl.ds(q * _FCH, _FCH)],
                fbuf.at[pl.ds(slot, _FCH)], fsem).wait()

            def grp(j, accs):
                lb0 = q * _FCH + j * 8 * _L
                fb0 = slot + j * 8 * _L
                out = []
                for u in range(8):
                    idx = lab_v[pl.ds(lb0 + u * _L, _L)]
                    g = plsc.load_gather(row_v, [idx])
                    dd = fbuf[pl.ds(fb0 + u * _L, _L)] - g
                    out.append(accs[u] + dd * dd)
                return tuple(out)

            accs = lax.fori_loop(0, _FCH // (8 * _L), grp, accs)

            @pl.when(q + 2 < 4)
            def _():
                pltpu.make_async_copy(
                    feat_hbm.at[d, pl.ds((q + 2) * _FCH, _FCH)],
                    fbuf.at[pl.ds(slot, _FCH)], fsem).start()

            return accs

        return lax.fori_loop(0, 4, quarter, accs)

    accs = lax.fori_loop(0, _DPW, dim_body, accs)

    part_v[...] = ((accs[0] + accs[1]) + (accs[2] + accs[3])) + (
        (accs[4] + accs[5]) + (accs[6] + accs[7]))
    pltpu.sync_copy(part_v, out_hbm.at[pl.ds(wid * _L, _L)])


_mesh = plsc.VectorSubcoreMesh(core_axis_name="c", subcore_axis_name="s")

_sc_call = functools.partial(
    pl.kernel,
    mesh=_mesh,
    compiler_params=pltpu.CompilerParams(needs_layout_passes=False),
    out_type=jax.ShapeDtypeStruct((_NW * _L,), jnp.float32),
    scratch_types=[
        pltpu.VMEM((_BATCH,), jnp.int32),
        pltpu.VMEM((_V,), jnp.float32),
        pltpu.VMEM((2 * _FCH,), jnp.float32),
        pltpu.VMEM((_L,), jnp.float32),
        pltpu.SemaphoreType.DMA,
        pltpu.SemaphoreType.DMA,
    ],
)(_tile_body)


def kernel(features, labels, centers):
    lab = labels.astype(jnp.int32)
    partials = _sc_call(features.T, lab, centers.T)
    return jnp.sum(partials) * 0.5
